# baseline (speedup 1.0000x reference)
import functools

import jax
import jax.numpy as jnp
from jax import lax
from jax.experimental import pallas as pl
from jax.experimental.pallas import tpu as pltpu
from jax.experimental.pallas import tpu_sc as plsc

_P = 5
_PR = _P // 2

_B, _C, _H, _W = 2, 96, 64, 128
_NC, _NS, _L = 2, 16, 16
_NW = _NC * _NS
_IMG = _H * _W

_SC_IMGS = 32
_TC_IMGS = _B * _C - _SC_IMGS


def _tc_body(x_ref, out_ref):
    x = x_ref[...]
    h = x.shape[1]
    row = jax.lax.broadcasted_iota(jnp.int32, (1, h, 1), 1)
    acc = jnp.zeros_like(x)
    nvalid = jnp.zeros((1, h, 1), dtype=x.dtype)
    for off in range(-_PR, _PR + 1):
        m = row + off
        ok = jnp.logical_and(m >= _PR, m <= h - 1 - _PR)
        acc = acc + jnp.where(ok, x, 0.0)
        nvalid = nvalid + ok.astype(x.dtype)
    acc = acc * jnp.array(_P, x.dtype)
    counting = nvalid * jnp.array(_P, x.dtype)
    out_ref[...] = acc / counting


def _tc_kernel(xtc):
    n = xtc.shape[0]
    bc_block = n // 2
    return pl.pallas_call(
        _tc_body,
        grid=(n // bc_block,),
        in_specs=[pl.BlockSpec((bc_block, _H, _W), lambda i: (i, 0, 0))],
        out_specs=pl.BlockSpec((bc_block, _H, _W), lambda i: (i, 0, 0)),
        out_shape=jax.ShapeDtypeStruct((n, _H, _W), jnp.float32),
    )(xtc)


_SC_TOTAL = _SC_IMGS * _IMG
_SC_CHUNK = _SC_TOTAL // _NW


def _sc_body(x_hbm, out_hbm, buf):
    wid = lax.axis_index("s") * _NC + lax.axis_index("c")
    base = wid * _SC_CHUNK
    pltpu.sync_copy(x_hbm.at[pl.ds(base, _SC_CHUNK)], buf)

    def step(i, carry):
        l = (i % (_IMG // _L)) // (_W // _L)
        v = buf[pl.ds(i * _L, _L)]
        acc = jnp.zeros((_L,), jnp.float32)
        nvalid = jnp.float32(0)
        for off in range(-_PR, _PR + 1):
            m = l + off
            ok = jnp.logical_and(m >= _PR, m <= _H - 1 - _PR)
            okf = ok.astype(jnp.float32)
            acc = acc + v * okf
            nvalid = nvalid + okf
        buf[pl.ds(i * _L, _L)] = acc * _P / (nvalid * _P)
        return carry

    lax.fori_loop(0, _SC_CHUNK // _L, step, 0, unroll=8)
    pltpu.sync_copy(buf, out_hbm.at[pl.ds(base, _SC_CHUNK)])


def _sc_kernel(xsc):
    run = pl.kernel(
        _sc_body,
        out_type=jax.ShapeDtypeStruct((_SC_TOTAL,), jnp.float32),
        scratch_types=[pltpu.VMEM((_SC_CHUNK,), jnp.float32)],
        mesh=plsc.VectorSubcoreMesh(core_axis_name="c", subcore_axis_name="s"),
    )
    return run(xsc)


def kernel(x):
    B, C, H, W = x.shape
    xf = x.reshape(B * C, H, W)
    out_tc = _tc_kernel(xf[:_TC_IMGS])
    out_sc = _sc_kernel(xf[_TC_IMGS:].reshape(_SC_TOTAL))
    out = jnp.concatenate([out_tc, out_sc.reshape(_SC_IMGS, H, W)], axis=0)
    return out.reshape(B, C, H, W)


# speedup vs baseline: 6.4398x; 6.4398x over previous
import jax
import jax.numpy as jnp
from jax.experimental import pallas as pl

_P = 5
_PR = _P // 2


def _overlap_add_body(x_ref, out_ref):
    x = x_ref[...]
    h = x.shape[1]
    row = jax.lax.broadcasted_iota(jnp.int32, (1, h, 1), 1)
    acc = jnp.zeros_like(x)
    nvalid = jnp.zeros((1, h, 1), dtype=x.dtype)
    for off in range(-_PR, _PR + 1):
        m = row + off
        ok = jnp.logical_and(m >= _PR, m <= h - 1 - _PR)
        acc = acc + jnp.where(ok, x, 0.0)
        nvalid = nvalid + ok.astype(x.dtype)
    acc = acc * jnp.array(_P, x.dtype)
    counting = nvalid * jnp.array(_P, x.dtype)
    out_ref[...] = acc / counting


def kernel(x):
    B, C, H, W = x.shape
    xf = x.reshape(B * C, H, W)
    bc_block = 96
    grid = (B * C) // bc_block
    out = pl.pallas_call(
        _overlap_add_body,
        grid=(grid,),
        in_specs=[pl.BlockSpec((bc_block, H, W), lambda i: (i, 0, 0))],
        out_specs=pl.BlockSpec((bc_block, H, W), lambda i: (i, 0, 0)),
        out_shape=jax.ShapeDtypeStruct((B * C, H, W), x.dtype),
    )(xf)
    return out.reshape(B, C, H, W)
